# baseline (speedup 1.0000x reference)
import functools

import jax
import jax.numpy as jnp
from jax import lax
from jax.experimental import pallas as pl
from jax.experimental.pallas import tpu as pltpu
from jax.experimental.pallas import tpu_sc as plsc

VOCAB = 1000
BATCH = 16384
DIM = 1000
DIM_PAD = 1024

_INFO = plsc.get_sparse_core_info()
NC = _INFO.num_cores
NS = _INFO.num_subcores
NW = NC * NS
B_PER_W = BATCH // NW
CHUNK = 64
NCHUNK = B_PER_W // CHUNK

_MESH = plsc.VectorSubcoreMesh(core_axis_name="c", subcore_axis_name="s")


@functools.partial(
    pl.kernel,
    mesh=_MESH,
    out_type=jax.ShapeDtypeStruct((BATCH, DIM), jnp.float32),
    scratch_types=[
        pltpu.VMEM((NCHUNK, CHUNK), jnp.int32),
        pltpu.VMEM((CHUNK, DIM), jnp.float32),
        pltpu.SemaphoreType.DMA,
    ],
    compiler_params=pltpu.CompilerParams(use_tc_tiling_on_sc=False),
)
def _gather_rows(idx_hbm, table_hbm, out_hbm, idx_v, rows_v, sem):
    wid = lax.axis_index("s") * NC + lax.axis_index("c")
    pltpu.sync_copy(idx_hbm.at[wid], idx_v)
    base = wid * B_PER_W

    def body(i, carry):
        pltpu.async_copy(table_hbm.at[idx_v.at[i]], rows_v, sem).wait()
        pltpu.sync_copy(rows_v, out_hbm.at[pl.ds(base + i * CHUNK, CHUNK)])
        return carry

    lax.fori_loop(0, NCHUNK, body, 0)


def kernel(idx, table):
    idx3 = idx.astype(jnp.int32).reshape(NW, NCHUNK, CHUNK)
    return _gather_rows(idx3, table)


# speedup vs baseline: 1.3376x; 1.3376x over previous
import functools

import jax
import jax.numpy as jnp
from jax import lax
from jax.experimental import pallas as pl
from jax.experimental.pallas import tpu as pltpu
from jax.experimental.pallas import tpu_sc as plsc

VOCAB = 1000
BATCH = 16384
DIM = 1000
DIM_PAD = 1024
DIM_MAIN = 896
DIM_TAIL = DIM - DIM_MAIN

_INFO = plsc.get_sparse_core_info()
NC = _INFO.num_cores
NS = _INFO.num_subcores
NW = NC * NS
B_PER_W = BATCH // NW
CHUNK = 64
NCHUNK = B_PER_W // CHUNK

_MESH = plsc.VectorSubcoreMesh(core_axis_name="c", subcore_axis_name="s")


@functools.partial(
    pl.kernel,
    mesh=_MESH,
    out_type=jax.ShapeDtypeStruct((BATCH, DIM), jnp.float32),
    scratch_types=[
        pltpu.VMEM((NCHUNK, CHUNK), jnp.int32),
        pltpu.VMEM((CHUNK, DIM_PAD), jnp.float32),
        pltpu.VMEM((CHUNK, DIM_TAIL), jnp.float32),
        pltpu.SemaphoreType.DMA,
    ],
)
def _gather_rows(idx_hbm, table_hbm, out_hbm, idx_v, rows_v, tail_v, sem):
    wid = lax.axis_index("s") * NC + lax.axis_index("c")
    pltpu.sync_copy(idx_hbm.at[wid], idx_v)
    base = wid * B_PER_W

    def body(i, carry):
        pltpu.async_copy(table_hbm.at[idx_v.at[i]], rows_v, sem).wait()
        rows = pl.ds(base + i * CHUNK, CHUNK)
        pltpu.sync_copy(rows_v.at[:, pl.ds(0, DIM_MAIN)],
                        out_hbm.at[rows, pl.ds(0, DIM_MAIN)])

        def tail_row(r, c2):
            for k in range(6):
                tail_v[r, pl.ds(k * 16, 16)] = rows_v[r, pl.ds(DIM_MAIN + k * 16, 16)]
            tail_v[r, pl.ds(DIM_TAIL - 16, 16)] = rows_v[r, pl.ds(DIM - 16, 16)]
            return c2

        lax.fori_loop(0, CHUNK, tail_row, 0)
        pltpu.sync_copy(tail_v, out_hbm.at[rows, pl.ds(DIM_MAIN, DIM_TAIL)])
        return carry

    lax.fori_loop(0, NCHUNK, body, 0)


def kernel(idx, table):
    idx3 = idx.astype(jnp.int32).reshape(NW, NCHUNK, CHUNK)
    table_pad = jnp.pad(table, ((0, 0), (0, DIM_PAD - DIM)))
    return _gather_rows(idx3, table_pad)


# speedup vs baseline: 1.4518x; 1.0854x over previous
import functools

import jax
import jax.numpy as jnp
from jax import lax
from jax.experimental import pallas as pl
from jax.experimental.pallas import tpu as pltpu
from jax.experimental.pallas import tpu_sc as plsc

VOCAB = 1000
BATCH = 16384
DIM = 1000
DIM_PAD = 1024
DIM_MAIN = 896
DIM_TAIL = DIM - DIM_MAIN

_INFO = plsc.get_sparse_core_info()
NC = _INFO.num_cores
NS = _INFO.num_subcores
NW = NC * NS
B_PER_W = BATCH // NW
CHUNK = 32
NCHUNK = B_PER_W // CHUNK

_MESH = plsc.VectorSubcoreMesh(core_axis_name="c", subcore_axis_name="s")


@functools.partial(
    pl.kernel,
    mesh=_MESH,
    out_type=jax.ShapeDtypeStruct((BATCH, DIM), jnp.float32),
    scratch_types=[
        pltpu.VMEM((NCHUNK, CHUNK), jnp.int32),
        pltpu.VMEM((CHUNK, DIM_PAD), jnp.float32),
        pltpu.VMEM((CHUNK, DIM_PAD), jnp.float32),
        pltpu.VMEM((CHUNK, DIM_TAIL), jnp.float32),
        pltpu.VMEM((CHUNK, DIM_TAIL), jnp.float32),
        pltpu.SemaphoreType.DMA,
        pltpu.SemaphoreType.DMA,
        pltpu.SemaphoreType.DMA,
        pltpu.SemaphoreType.DMA,
    ],
)
def _gather_rows(idx_hbm, table_hbm, out_hbm, idx_v,
                 rows_v0, rows_v1, tail_v0, tail_v1,
                 sem_g0, sem_g1, sem_o0, sem_o1):
    wid = lax.axis_index("s") * NC + lax.axis_index("c")
    pltpu.sync_copy(idx_hbm.at[wid], idx_v)
    base = wid * B_PER_W

    bufs = (rows_v0, rows_v1)
    tails = (tail_v0, tail_v1)
    sem_g = (sem_g0, sem_g1)
    sem_o = (sem_o0, sem_o1)

    gathers = [None, None]
    outs = [None, None]

    gathers[0] = pltpu.async_copy(table_hbm.at[idx_v.at[0]], bufs[0], sem_g[0])
    for i in range(NCHUNK):
        b = i % 2
        nb = (i + 1) % 2
        if i + 1 < NCHUNK:
            if outs[nb] is not None:
                for o in outs[nb]:
                    o.wait()
                outs[nb] = None
            gathers[nb] = pltpu.async_copy(
                table_hbm.at[idx_v.at[i + 1]], bufs[nb], sem_g[nb])
        gathers[b].wait()

        def tail_row(r, c2, _b=b):
            for k in range(6):
                tails[_b][r, pl.ds(k * 16, 16)] = \
                    bufs[_b][r, pl.ds(DIM_MAIN + k * 16, 16)]
            tails[_b][r, pl.ds(DIM_TAIL - 16, 16)] = bufs[_b][r, pl.ds(DIM - 16, 16)]
            return c2

        lax.fori_loop(0, CHUNK, tail_row, 0)

        rows = pl.ds(base + i * CHUNK, CHUNK)
        outs[b] = (
            pltpu.async_copy(bufs[b].at[:, pl.ds(0, DIM_MAIN)],
                             out_hbm.at[rows, pl.ds(0, DIM_MAIN)], sem_o[b]),
            pltpu.async_copy(tails[b],
                             out_hbm.at[rows, pl.ds(DIM_MAIN, DIM_TAIL)], sem_o[b]),
        )

    for pair in outs:
        if pair is not None:
            for o in pair:
                o.wait()


def kernel(idx, table):
    idx3 = idx.astype(jnp.int32).reshape(NW, NCHUNK, CHUNK)
    table_pad = jnp.pad(table, ((0, 0), (0, DIM_PAD - DIM)))
    return _gather_rows(idx3, table_pad)
